# Initial kernel scaffold; baseline (speedup 1.0000x reference)
import jax
import jax.numpy as jnp
from jax.experimental import pallas as pl


def kernel(x, params, edge_index, ptr, batch):
    raise NotImplementedError("write your pallas kernel here")


# speedup vs baseline: 17.6490x; 17.6490x over previous
import functools

import jax
import jax.numpy as jnp
from jax import lax
from jax.experimental import pallas as pl
from jax.experimental.pallas import tpu as pltpu
from jax.experimental.pallas import tpu_sc as plsc

N = 100000
E = 3200000
G = 1000
D = 16
W = 50
NPD = N // G

_LANES = 128
_CH = 8
_TILES = 32
_ROWS_PER_TILE = 792
_STEPS = _ROWS_PER_TILE // _CH
_EROWS = _TILES * _ROWS_PER_TILE
_EPAD = _EROWS * _LANES
_ZROWS = 6272
_NACC = 16 * _ZROWS


def _sc_body(xproc, srcp, dstp, zz, out, acc, srcv, dstv, rows, sem):
    c = lax.axis_index("c")
    s = lax.axis_index("s")
    tid = c * 16 + s
    pltpu.sync_copy(zz, acc.at[pl.ds(s * _ZROWS, _ZROWS)])
    plsc.subcore_barrier()

    def step(i, carry):
        row0 = tid * _ROWS_PER_TILE + i * _CH
        pltpu.sync_copy(srcp.at[pl.ds(row0, _CH)], srcv)
        pltpu.sync_copy(dstp.at[pl.ds(row0, _CH)], dstv)
        cps = [
            pltpu.async_copy(xproc.at[srcv.at[j]], rows.at[j], sem)
            for j in range(_CH)
        ]
        for cp in cps:
            cp.wait()
        for j in range(_CH):
            pltpu.sync_copy(rows.at[j], acc.at[dstv.at[j]], add=True)
        return carry

    lax.fori_loop(0, _STEPS, step, 0)
    plsc.subcore_barrier()
    pltpu.sync_copy(
        acc.at[pl.ds(s * _ZROWS, _ZROWS)], out.at[c, pl.ds(s * _ZROWS, _ZROWS)]
    )


def _edge_partials(xproc, srcp, dstp, zz):
    f = pl.kernel(
        _sc_body,
        out_type=jax.ShapeDtypeStruct((2, _NACC, D), jnp.float32),
        mesh=plsc.VectorSubcoreMesh(core_axis_name="c", subcore_axis_name="s"),
        scratch_types=[
            pltpu.VMEM_SHARED((_NACC, D), jnp.float32),
            pltpu.VMEM((_CH, _LANES), jnp.int32),
            pltpu.VMEM((_CH, _LANES), jnp.int32),
            pltpu.VMEM((_CH, _LANES, D), jnp.float32),
            pltpu.SemaphoreType.DMA,
        ],
        compiler_params=pltpu.CompilerParams(use_tc_tiling_on_sc=False),
    )
    return f(xproc, srcp, dstp, zz)


_R = 1000
_NBLK = N // _R
_DBLK = _R // NPD


def _mlp3(h, w0, b0, w1, b1, w2, b2):
    h = jnp.tanh(jnp.dot(h, w0, preferred_element_type=jnp.float32) + b0)
    h = jnp.tanh(jnp.dot(h, w1, preferred_element_type=jnp.float32) + b1)
    return jnp.dot(h, w2, preferred_element_type=jnp.float32) + b2


def _full_spec(a):
    r = a.ndim
    return pl.BlockSpec(a.shape, lambda i, _r=r: (0,) * _r)


def _prep_proc(nf, pp, qq):
    def body(nf_r, pw0, pb0, pw1, pb1, pw2, pb2, qw0, qb0, qw1, qb1, qw2, qb2,
             prep_o, proc_o):
        xprep = _mlp3(nf_r[...], pw0[...], pb0[...], pw1[...], pb1[...],
                      pw2[...], pb2[...])
        prep_o[...] = xprep
        proc_o[...] = _mlp3(xprep, qw0[...], qb0[...], qw1[...], qb1[...],
                            qw2[...], qb2[...])

    return pl.pallas_call(
        body,
        grid=(_NBLK,),
        in_specs=[pl.BlockSpec((_R, 2), lambda i: (i, 0))]
        + [_full_spec(a) for a in pp] + [_full_spec(a) for a in qq],
        out_specs=[pl.BlockSpec((_R, D), lambda i: (i, 0))] * 2,
        out_shape=[jax.ShapeDtypeStruct((N, D), jnp.float32)] * 2,
    )(nf, *pp, *qq)


def _agg_stage(partials, xprep, nf, aa):
    def body(pa, pb, xprep_r, nf_r, w0, b0, w1, b1, w2, b2, nemb_o, dsum_o):
        aggr = pa[0] + pb[0]
        xagg = _mlp3(aggr, w0[...], b0[...], w1[...], b1[...], w2[...], b2[...])
        nemb = xprep_r[...] + xagg
        nemb_o[...] = nemb
        ncomb = jnp.concatenate([nf_r[...], nemb], axis=1)
        rr = lax.broadcasted_iota(jnp.int32, (_DBLK, _R), 1) // NPD
        dd = lax.broadcasted_iota(jnp.int32, (_DBLK, _R), 0)
        sel = (rr == dd).astype(jnp.float32)
        dsum = jnp.dot(sel, ncomb, preferred_element_type=jnp.float32)
        dsum_o[...] = dsum.reshape(1, _DBLK, D + 2)

    return pl.pallas_call(
        body,
        grid=(_NBLK,),
        in_specs=[
            pl.BlockSpec((1, _R, D), lambda i: (0, i, 0)),
            pl.BlockSpec((1, _R, D), lambda i: (1, i, 0)),
            pl.BlockSpec((_R, D), lambda i: (i, 0)),
            pl.BlockSpec((_R, 2), lambda i: (i, 0)),
        ] + [_full_spec(a) for a in aa],
        out_specs=[
            pl.BlockSpec((_R, D), lambda i: (i, 0)),
            pl.BlockSpec((1, _DBLK, D + 2), lambda i: (i, 0, 0)),
        ],
        out_shape=[
            jax.ShapeDtypeStruct((N, D), jnp.float32),
            jax.ShapeDtypeStruct((_NBLK, _DBLK, D + 2), jnp.float32),
        ],
    )(partials, partials, xprep, nf, *aa)


def _dag_glob(dagfeat, dagsum, x00, dd, gg):
    def body(df, ds_, x0, dw0, db0, dw1, db1, dw2, db2,
             gw0, gb0, gw1, gb1, gw2, gb2, demb_o, glob_o):
        din = jnp.concatenate([df[...], ds_[...]], axis=1)
        demb = _mlp3(din, dw0[...], db0[...], dw1[...], db1[...], dw2[...],
                     db2[...])
        demb_o[...] = demb
        dagg = jnp.sum(demb, axis=0, keepdims=True)
        gin = jnp.concatenate([x0[...], dagg], axis=1)
        glob_o[...] = _mlp3(gin, gw0[...], gb0[...], gw1[...], gb1[...],
                            gw2[...], gb2[...])

    return pl.pallas_call(
        body,
        grid=(1,),
        in_specs=[_full_spec(a) for a in
                  (dagfeat, dagsum, x00, *dd, *gg)],
        out_specs=[
            pl.BlockSpec((G, D), lambda i: (0, 0)),
            pl.BlockSpec((1, D), lambda i: (0, 0)),
        ],
        out_shape=[
            jax.ShapeDtypeStruct((G, D), jnp.float32),
            jax.ShapeDtypeStruct((1, D), jnp.float32),
        ],
    )(dagfeat, dagsum, x00, *dd, *gg)


_DGB = 200
_DGRID = G // _DGB
_DROWS = _DGB * W


def _dag_scores(demb, glob, uu):
    def body(de, gl, w0, b0, w1, b1, w2, b2, out_o):
        rr = lax.broadcasted_iota(jnp.int32, (_DROWS, _DGB), 0) // W
        dd = lax.broadcasted_iota(jnp.int32, (_DROWS, _DGB), 1)
        sel = (rr == dd).astype(jnp.float32)
        drpt = jnp.dot(sel, de[...], preferred_element_type=jnp.float32)
        gb = jnp.broadcast_to(gl[...], (_DROWS, D))
        wcol = (lax.broadcasted_iota(jnp.int32, (_DROWS, 1), 0) % W).astype(
            jnp.float32)
        din = jnp.concatenate([drpt, gb, wcol], axis=1)
        out_o[...] = _mlp3(din, w0[...], b0[...], w1[...], b1[...], w2[...],
                           b2[...])

    return pl.pallas_call(
        body,
        grid=(_DGRID,),
        in_specs=[pl.BlockSpec((_DGB, D), lambda i: (i, 0)),
                  _full_spec(glob)] + [_full_spec(a) for a in uu],
        out_specs=pl.BlockSpec((_DROWS, 1), lambda i: (i, 0)),
        out_shape=jax.ShapeDtypeStruct((G * W, 1), jnp.float32),
    )(demb, glob, *uu)


def _node_scores(nemb, demb3, glob, vv):
    def body(ne, de, gl, w0, b0, w1, b1, w2, b2, out_o):
        rr = lax.broadcasted_iota(jnp.int32, (_R, _DBLK), 0) // NPD
        dd = lax.broadcasted_iota(jnp.int32, (_R, _DBLK), 1)
        sel = (rr == dd).astype(jnp.float32)
        drpt = jnp.dot(sel, de[...][0], preferred_element_type=jnp.float32)
        gb = jnp.broadcast_to(gl[...], (_R, D))
        nin = jnp.concatenate([ne[...], drpt, gb], axis=1)
        out_o[...] = _mlp3(nin, w0[...], b0[...], w1[...], b1[...], w2[...],
                           b2[...])

    return pl.pallas_call(
        body,
        grid=(_NBLK,),
        in_specs=[
            pl.BlockSpec((_R, D), lambda i: (i, 0)),
            pl.BlockSpec((1, _DBLK, D), lambda i: (i, 0, 0)),
            _full_spec(glob),
        ] + [_full_spec(a) for a in vv],
        out_specs=pl.BlockSpec((_R, 1), lambda i: (i, 0)),
        out_shape=jax.ShapeDtypeStruct((N, 1), jnp.float32),
    )(nemb, demb3, glob, *vv)


def _p(ps):
    return (ps[0], ps[1].reshape(1, -1), ps[2], ps[3].reshape(1, -1),
            ps[4], ps[5].reshape(1, -1))


def kernel(x, params, edge_index, ptr, batch):
    nf = x[:, 3:5]
    x00 = x[0, 0].reshape(1, 1)
    dagfeat = x.reshape(G, NPD, 5)[:, 0, 1:3]

    src = edge_index[0]
    dst = edge_index[1]
    srcp = jnp.concatenate(
        [src, jnp.zeros((_EPAD - E,), jnp.int32)]).reshape(_EROWS, _LANES)
    dstp = jnp.concatenate(
        [dst, jnp.full((_EPAD - E,), N, jnp.int32)]).reshape(_EROWS, _LANES)
    zz = jnp.zeros((_ZROWS, D), jnp.float32)

    xprep, xproc = _prep_proc(nf, _p(params['prep']), _p(params['proc']))
    partials = _edge_partials(xproc, srcp, dstp, zz)
    nemb, dagsum3 = _agg_stage(partials, xprep, nf, _p(params['agg']))
    demb, glob = _dag_glob(dagfeat, dagsum3.reshape(G, D + 2), x00,
                           _p(params['dag']), _p(params['glob']))
    dsc = _dag_scores(demb, glob, _p(params['dag_score']))
    nsc = _node_scores(nemb, demb.reshape(_NBLK, _DBLK, D), glob,
                       _p(params['node_score']))
    return nsc[:, 0], dsc.reshape(G, W)


# speedup vs baseline: 20.4409x; 1.1582x over previous
import functools

import jax
import jax.numpy as jnp
from jax import lax
from jax.experimental import pallas as pl
from jax.experimental.pallas import tpu as pltpu
from jax.experimental.pallas import tpu_sc as plsc

N = 100000
E = 3200000
G = 1000
D = 16
W = 50
NPD = N // G

_LANES = 128
_CH = 6
_CE = _CH * _LANES
_TILES = 32
_ROWS_PER_TILE = 792
_STEPS = _ROWS_PER_TILE // _CH
_PAIRS = _STEPS // 2
_EROWS = _TILES * _ROWS_PER_TILE
_EPAD = _EROWS * _LANES
_ZROWS = 6256
_NACC = 16 * _ZROWS


def _sc_body(xproc, srcp, dstp, zz, out, acc, srcv, dstv, rows, gsem, ssem,
             isem):
    c = lax.axis_index("c")
    s = lax.axis_index("s")
    tid = c * 16 + s
    base = tid * _ROWS_PER_TILE
    pltpu.sync_copy(zz, acc.at[pl.ds(s * _ZROWS, _ZROWS)])
    plsc.subcore_barrier()

    def fire_gathers(b):
        for j in range(_CH):
            pltpu.async_copy(
                xproc.at[srcv.at[b, j]],
                rows.at[b, pl.ds(j * _LANES, _LANES)], gsem)

    def drain_gathers(b):
        for j in range(_CH):
            pltpu.make_async_copy(
                xproc.at[srcv.at[b, j]],
                rows.at[b, pl.ds(j * _LANES, _LANES)], gsem).wait()

    def fire_scatters(b):
        for j in range(_CH):
            pltpu.async_copy(
                rows.at[b, pl.ds(j * _LANES, _LANES)],
                acc.at[dstv.at[b, j]], ssem, add=True)

    def drain_scatters(b):
        for j in range(_CH):
            pltpu.make_async_copy(
                rows.at[b, pl.ds(j * _LANES, _LANES)],
                acc.at[dstv.at[b, j]], ssem).wait()

    def issue_idx(row, b):
        pltpu.async_copy(srcp.at[pl.ds(row, _CH)], srcv.at[b], isem)
        pltpu.async_copy(dstp.at[pl.ds(row, _CH)], dstv.at[b], isem)

    def wait_idx(b):
        pltpu.make_async_copy(srcp.at[pl.ds(0, _CH)], srcv.at[b], isem).wait()
        pltpu.make_async_copy(dstp.at[pl.ds(0, _CH)], dstv.at[b], isem).wait()

    pltpu.sync_copy(zz.at[pl.ds(0, _CE)], rows.at[1])
    pltpu.sync_copy(srcp.at[pl.ds(base + _CH, _CH)], srcv.at[1])
    pltpu.sync_copy(dstp.at[pl.ds(base + _CH, _CH)], dstv.at[1])
    fire_scatters(1)
    issue_idx(base, 0)

    def pair(i, carry):
        for b in (0, 1):
            g = 2 * i + b
            wait_idx(b)
            fire_gathers(b)
            drain_scatters(1 - b)
            nxt = jnp.minimum(g + 1, _STEPS - 1) * _CH
            issue_idx(base + nxt, 1 - b)
            drain_gathers(b)
            fire_scatters(b)
        return carry

    lax.fori_loop(0, _PAIRS, pair, 0)
    drain_scatters(1)
    wait_idx(0)
    plsc.subcore_barrier()
    pltpu.sync_copy(
        acc.at[pl.ds(s * _ZROWS, _ZROWS)], out.at[c, pl.ds(s * _ZROWS, _ZROWS)]
    )


def _edge_partials(xproc, srcp, dstp, zz):
    f = pl.kernel(
        _sc_body,
        out_type=jax.ShapeDtypeStruct((2, _NACC, D), jnp.float32),
        mesh=plsc.VectorSubcoreMesh(core_axis_name="c", subcore_axis_name="s"),
        scratch_types=[
            pltpu.VMEM_SHARED((_NACC, D), jnp.float32),
            pltpu.VMEM((2, _CH, _LANES), jnp.int32),
            pltpu.VMEM((2, _CH, _LANES), jnp.int32),
            pltpu.VMEM((2, _CE, D), jnp.float32),
            pltpu.SemaphoreType.DMA,
            pltpu.SemaphoreType.DMA,
            pltpu.SemaphoreType.DMA,
        ],
        compiler_params=pltpu.CompilerParams(use_tc_tiling_on_sc=False),
    )
    return f(xproc, srcp, dstp, zz)


_R = 1000
_NBLK = N // _R
_DBLK = _R // NPD


def _mlp3(h, w0, b0, w1, b1, w2, b2):
    h = jnp.tanh(jnp.dot(h, w0, preferred_element_type=jnp.float32) + b0)
    h = jnp.tanh(jnp.dot(h, w1, preferred_element_type=jnp.float32) + b1)
    return jnp.dot(h, w2, preferred_element_type=jnp.float32) + b2


def _full_spec(a):
    r = a.ndim
    return pl.BlockSpec(a.shape, lambda i, _r=r: (0,) * _r)


def _prep_proc(nf, pp, qq):
    def body(nf_r, pw0, pb0, pw1, pb1, pw2, pb2, qw0, qb0, qw1, qb1, qw2, qb2,
             prep_o, proc_o):
        xprep = _mlp3(nf_r[...], pw0[...], pb0[...], pw1[...], pb1[...],
                      pw2[...], pb2[...])
        prep_o[...] = xprep
        proc_o[...] = _mlp3(xprep, qw0[...], qb0[...], qw1[...], qb1[...],
                            qw2[...], qb2[...])

    return pl.pallas_call(
        body,
        grid=(_NBLK,),
        in_specs=[pl.BlockSpec((_R, 2), lambda i: (i, 0))]
        + [_full_spec(a) for a in pp] + [_full_spec(a) for a in qq],
        out_specs=[pl.BlockSpec((_R, D), lambda i: (i, 0))] * 2,
        out_shape=[jax.ShapeDtypeStruct((N, D), jnp.float32)] * 2,
    )(nf, *pp, *qq)


def _agg_stage(partials, xprep, nf, aa):
    def body(pa, pb, xprep_r, nf_r, w0, b0, w1, b1, w2, b2, nemb_o, dsum_o):
        aggr = pa[0] + pb[0]
        xagg = _mlp3(aggr, w0[...], b0[...], w1[...], b1[...], w2[...], b2[...])
        nemb = xprep_r[...] + xagg
        nemb_o[...] = nemb
        ncomb = jnp.concatenate([nf_r[...], nemb], axis=1)
        rr = lax.broadcasted_iota(jnp.int32, (_DBLK, _R), 1) // NPD
        dd = lax.broadcasted_iota(jnp.int32, (_DBLK, _R), 0)
        sel = (rr == dd).astype(jnp.float32)
        dsum = jnp.dot(sel, ncomb, preferred_element_type=jnp.float32)
        dsum_o[...] = dsum.reshape(1, _DBLK, D + 2)

    return pl.pallas_call(
        body,
        grid=(_NBLK,),
        in_specs=[
            pl.BlockSpec((1, _R, D), lambda i: (0, i, 0)),
            pl.BlockSpec((1, _R, D), lambda i: (1, i, 0)),
            pl.BlockSpec((_R, D), lambda i: (i, 0)),
            pl.BlockSpec((_R, 2), lambda i: (i, 0)),
        ] + [_full_spec(a) for a in aa],
        out_specs=[
            pl.BlockSpec((_R, D), lambda i: (i, 0)),
            pl.BlockSpec((1, _DBLK, D + 2), lambda i: (i, 0, 0)),
        ],
        out_shape=[
            jax.ShapeDtypeStruct((N, D), jnp.float32),
            jax.ShapeDtypeStruct((_NBLK, _DBLK, D + 2), jnp.float32),
        ],
    )(partials, partials, xprep, nf, *aa)


def _dag_glob(dagfeat, dagsum, x00, dd, gg):
    def body(df, ds_, x0, dw0, db0, dw1, db1, dw2, db2,
             gw0, gb0, gw1, gb1, gw2, gb2, demb_o, glob_o):
        din = jnp.concatenate([df[...], ds_[...]], axis=1)
        demb = _mlp3(din, dw0[...], db0[...], dw1[...], db1[...], dw2[...],
                     db2[...])
        demb_o[...] = demb
        dagg = jnp.sum(demb, axis=0, keepdims=True)
        gin = jnp.concatenate([x0[...], dagg], axis=1)
        glob_o[...] = _mlp3(gin, gw0[...], gb0[...], gw1[...], gb1[...],
                            gw2[...], gb2[...])

    return pl.pallas_call(
        body,
        grid=(1,),
        in_specs=[_full_spec(a) for a in
                  (dagfeat, dagsum, x00, *dd, *gg)],
        out_specs=[
            pl.BlockSpec((G, D), lambda i: (0, 0)),
            pl.BlockSpec((1, D), lambda i: (0, 0)),
        ],
        out_shape=[
            jax.ShapeDtypeStruct((G, D), jnp.float32),
            jax.ShapeDtypeStruct((1, D), jnp.float32),
        ],
    )(dagfeat, dagsum, x00, *dd, *gg)


_DGB = 200
_DGRID = G // _DGB
_DROWS = _DGB * W


def _dag_scores(demb, glob, uu):
    def body(de, gl, w0, b0, w1, b1, w2, b2, out_o):
        rr = lax.broadcasted_iota(jnp.int32, (_DROWS, _DGB), 0) // W
        dd = lax.broadcasted_iota(jnp.int32, (_DROWS, _DGB), 1)
        sel = (rr == dd).astype(jnp.float32)
        drpt = jnp.dot(sel, de[...], preferred_element_type=jnp.float32)
        gb = jnp.broadcast_to(gl[...], (_DROWS, D))
        wcol = (lax.broadcasted_iota(jnp.int32, (_DROWS, 1), 0) % W).astype(
            jnp.float32)
        din = jnp.concatenate([drpt, gb, wcol], axis=1)
        out_o[...] = _mlp3(din, w0[...], b0[...], w1[...], b1[...], w2[...],
                           b2[...])

    return pl.pallas_call(
        body,
        grid=(_DGRID,),
        in_specs=[pl.BlockSpec((_DGB, D), lambda i: (i, 0)),
                  _full_spec(glob)] + [_full_spec(a) for a in uu],
        out_specs=pl.BlockSpec((_DROWS, 1), lambda i: (i, 0)),
        out_shape=jax.ShapeDtypeStruct((G * W, 1), jnp.float32),
    )(demb, glob, *uu)


def _node_scores(nemb, demb3, glob, vv):
    def body(ne, de, gl, w0, b0, w1, b1, w2, b2, out_o):
        rr = lax.broadcasted_iota(jnp.int32, (_R, _DBLK), 0) // NPD
        dd = lax.broadcasted_iota(jnp.int32, (_R, _DBLK), 1)
        sel = (rr == dd).astype(jnp.float32)
        drpt = jnp.dot(sel, de[...][0], preferred_element_type=jnp.float32)
        gb = jnp.broadcast_to(gl[...], (_R, D))
        nin = jnp.concatenate([ne[...], drpt, gb], axis=1)
        out_o[...] = _mlp3(nin, w0[...], b0[...], w1[...], b1[...], w2[...],
                           b2[...])

    return pl.pallas_call(
        body,
        grid=(_NBLK,),
        in_specs=[
            pl.BlockSpec((_R, D), lambda i: (i, 0)),
            pl.BlockSpec((1, _DBLK, D), lambda i: (i, 0, 0)),
            _full_spec(glob),
        ] + [_full_spec(a) for a in vv],
        out_specs=pl.BlockSpec((_R, 1), lambda i: (i, 0)),
        out_shape=jax.ShapeDtypeStruct((N, 1), jnp.float32),
    )(nemb, demb3, glob, *vv)


def _p(ps):
    return (ps[0], ps[1].reshape(1, -1), ps[2], ps[3].reshape(1, -1),
            ps[4], ps[5].reshape(1, -1))


def kernel(x, params, edge_index, ptr, batch):
    nf = x[:, 3:5]
    x00 = x[0, 0].reshape(1, 1)
    dagfeat = x.reshape(G, NPD, 5)[:, 0, 1:3]

    src = edge_index[0]
    dst = edge_index[1]
    srcp = jnp.concatenate(
        [src, jnp.zeros((_EPAD - E,), jnp.int32)]).reshape(_EROWS, _LANES)
    dstp = jnp.concatenate(
        [dst, jnp.full((_EPAD - E,), N, jnp.int32)]).reshape(_EROWS, _LANES)
    zz = jnp.zeros((_ZROWS, D), jnp.float32)

    xprep, xproc = _prep_proc(nf, _p(params['prep']), _p(params['proc']))
    partials = _edge_partials(xproc, srcp, dstp, zz)
    nemb, dagsum3 = _agg_stage(partials, xprep, nf, _p(params['agg']))
    demb, glob = _dag_glob(dagfeat, dagsum3.reshape(G, D + 2), x00,
                           _p(params['dag']), _p(params['glob']))
    dsc = _dag_scores(demb, glob, _p(params['dag_score']))
    nsc = _node_scores(nemb, demb.reshape(_NBLK, _DBLK, D), glob,
                       _p(params['node_score']))
    return nsc[:, 0], dsc.reshape(G, W)


# speedup vs baseline: 22.0478x; 1.0786x over previous
import numpy as np

import jax
import jax.numpy as jnp
from jax import lax
from jax.experimental import pallas as pl
from jax.experimental.pallas import tpu as pltpu
from jax.experimental.pallas import tpu_sc as plsc

N = 100000
E = 3200000
G = 1000
D = 16
W = 50
NPD = N // G

_LANES = 128
_CH = 6
_CE = _CH * _LANES
_TILES = 32
_ROWS_PER_TILE = 792
_STEPS = _ROWS_PER_TILE // _CH
_PAIRS = _STEPS // 2
_EROWS = _TILES * _ROWS_PER_TILE
_EPAD = _EROWS * _LANES
_ZROWS = 6256
_NACC = 16 * _ZROWS


def _sc_body(xproc, srcp, dstp, zz, out, acc, srcv, dstv, rows, gsem, ssem,
             isem):
    c = lax.axis_index("c")
    s = lax.axis_index("s")
    tid = c * 16 + s
    base = tid * _ROWS_PER_TILE
    pltpu.sync_copy(zz, acc.at[pl.ds(s * _ZROWS, _ZROWS)])
    plsc.subcore_barrier()

    def fire_gathers(b):
        for j in range(_CH):
            pltpu.async_copy(
                xproc.at[srcv.at[b, j]],
                rows.at[b, pl.ds(j * _LANES, _LANES)], gsem)

    def drain_gathers(b):
        for j in range(_CH):
            pltpu.make_async_copy(
                xproc.at[srcv.at[b, j]],
                rows.at[b, pl.ds(j * _LANES, _LANES)], gsem).wait()

    def fire_scatters(b):
        for j in range(_CH):
            pltpu.async_copy(
                rows.at[b, pl.ds(j * _LANES, _LANES)],
                acc.at[dstv.at[b, j]], ssem, add=True)

    def drain_scatters(b):
        for j in range(_CH):
            pltpu.make_async_copy(
                rows.at[b, pl.ds(j * _LANES, _LANES)],
                acc.at[dstv.at[b, j]], ssem).wait()

    def issue_idx(row, b):
        pltpu.async_copy(srcp.at[pl.ds(row, _CH)], srcv.at[b], isem)
        pltpu.async_copy(dstp.at[pl.ds(row, _CH)], dstv.at[b], isem)

    def wait_idx(b):
        pltpu.make_async_copy(srcp.at[pl.ds(0, _CH)], srcv.at[b], isem).wait()
        pltpu.make_async_copy(dstp.at[pl.ds(0, _CH)], dstv.at[b], isem).wait()

    pltpu.sync_copy(zz.at[pl.ds(0, _CE)], rows.at[1])
    pltpu.sync_copy(srcp.at[pl.ds(base + _CH, _CH)], srcv.at[1])
    pltpu.sync_copy(dstp.at[pl.ds(base + _CH, _CH)], dstv.at[1])
    fire_scatters(1)
    issue_idx(base, 0)

    def pair(i, carry):
        for b in (0, 1):
            g = 2 * i + b
            wait_idx(b)
            fire_gathers(b)
            drain_scatters(1 - b)
            nxt = jnp.minimum(g + 1, _STEPS - 1) * _CH
            issue_idx(base + nxt, 1 - b)
            drain_gathers(b)
            fire_scatters(b)
        return carry

    lax.fori_loop(0, _PAIRS, pair, 0)
    drain_scatters(1)
    wait_idx(0)
    plsc.subcore_barrier()
    pltpu.sync_copy(
        acc.at[pl.ds(s * _ZROWS, _ZROWS)], out.at[c, pl.ds(s * _ZROWS, _ZROWS)]
    )


def _edge_partials(xproc, srcp, dstp, zz):
    f = pl.kernel(
        _sc_body,
        out_type=jax.ShapeDtypeStruct((2, _NACC, D), jnp.float32),
        mesh=plsc.VectorSubcoreMesh(core_axis_name="c", subcore_axis_name="s"),
        scratch_types=[
            pltpu.VMEM_SHARED((_NACC, D), jnp.float32),
            pltpu.VMEM((2, _CH, _LANES), jnp.int32),
            pltpu.VMEM((2, _CH, _LANES), jnp.int32),
            pltpu.VMEM((2, _CE, D), jnp.float32),
            pltpu.SemaphoreType.DMA,
            pltpu.SemaphoreType.DMA,
            pltpu.SemaphoreType.DMA,
        ],
        compiler_params=pltpu.CompilerParams(use_tc_tiling_on_sc=False),
    )
    return f(xproc, srcp, dstp, zz)


_R = 4000
_PR = _R // 8
_NBLK = N // _R
_DBLK = _R // NPD
_NP = N // 8
_ACCP = _NACC // 8

_rr = np.arange(_PR)
_dlo = (8 * _rr) // NPD
_dhi = (8 * _rr + 7) // NPD
_bnd = _dlo != _dhi
_dd = np.arange(_DBLK)
_SFULL = ((~_bnd)[None, :] & (_dlo[None, :] == _dd[:, None])).astype(np.float32)
_SLOB = (_bnd[None, :] & (_dlo[None, :] == _dd[:, None])).astype(np.float32)
_SHIB = (_bnd[None, :] & (_dhi[None, :] == _dd[:, None])).astype(np.float32)
_ALLS = (_dlo[:, None] == _dd[None, :]).astype(np.float32)
_HIS = (_bnd[:, None] & (_dhi[:, None] == _dd[None, :])).astype(np.float32)
_lane = np.arange(128)
_M1 = (1.0 - (_bnd[:, None] & (_lane[None, :] >= 64))).astype(np.float32)
_M2 = (_bnd[:, None] & (_lane[None, :] >= 64)).astype(np.float32)
_MLO128 = (_lane < 64).astype(np.float32)[None, :]
_MHI128 = (_lane >= 64).astype(np.float32)[None, :]
_l16 = np.arange(16)
_MLO16 = (_l16 < 8).astype(np.float32)[None, :]
_MHI16 = (_l16 >= 8).astype(np.float32)[None, :]
_P8 = (_lane[:, None] % 16 == _l16[None, :]).astype(np.float32)
_P2 = (_l16[:, None] % 2 == np.arange(2)[None, :]).astype(np.float32)
_SEL35 = np.zeros((40, 16), np.float32)
for _s in range(8):
    _SEL35[5 * _s + 3, 2 * _s] = 1.0
    _SEL35[5 * _s + 4, 2 * _s + 1] = 1.0


def _bdot(a, b):
    return jnp.dot(a.astype(jnp.bfloat16), b.astype(jnp.bfloat16),
                   preferred_element_type=jnp.float32)


def _hdot(a, b):
    return jnp.dot(a, b, preferred_element_type=jnp.float32,
                   precision=jax.lax.Precision.HIGHEST)


def _mlp3(h, w0, b0, w1, b1, w2, b2):
    h = jnp.tanh(_bdot(h, w0) + b0)
    h = jnp.tanh(_bdot(h, w1) + b1)
    return _bdot(h, w2) + b2


def _bd8(w):
    return jnp.kron(jnp.eye(8, dtype=jnp.float32), w)


def _t8(b):
    return jnp.tile(b.reshape(1, -1), (1, 8))


def _pk(ps):
    return (_bd8(ps[0]), _t8(ps[1]), _bd8(ps[2]), _t8(ps[3]),
            _bd8(ps[4]), _t8(ps[5]))


def _full_spec(a):
    r = a.ndim
    return pl.BlockSpec(a.shape, lambda i, _r=r: (0,) * _r)


def _prep_proc(xp, sel35, pp, qq):
    def body(x_r, s35, pw0, pb0, pw1, pb1, pw2, pb2,
             qw0, qb0, qw1, qb1, qw2, qb2, prep_o, proc_o):
        nfp = _hdot(x_r[...][0], s35[...])
        xprep = _mlp3(nfp, pw0[...], pb0[...], pw1[...], pb1[...],
                      pw2[...], pb2[...])
        prep_o[...] = xprep.reshape(1, _PR, 128)
        proc = _mlp3(xprep, qw0[...], qb0[...], qw1[...], qb1[...],
                     qw2[...], qb2[...])
        proc_o[...] = proc.reshape(1, _PR, 128)

    return pl.pallas_call(
        body,
        grid=(_NBLK,),
        in_specs=[pl.BlockSpec((1, _PR, 40), lambda i: (i, 0, 0)),
                  _full_spec(sel35)]
        + [_full_spec(a) for a in pp] + [_full_spec(a) for a in qq],
        out_specs=[pl.BlockSpec((1, _PR, 128), lambda i: (i, 0, 0))] * 2,
        out_shape=[jax.ShapeDtypeStruct((_NBLK, _PR, 128), jnp.float32)] * 2,
    )(xp, sel35, *pp, *qq)


def _agg_stage(partials, xprep, nf16, aa, cs):
    def body(pa, pb, xprep_r, nf_r, w0, b0, w1, b1, w2, b2,
             sfull, slob, shib, p8, p2, mlo128, mhi128, mlo16, mhi16,
             nemb_o, dsne_o, dsnf_o):
        aggr = pa[0, 0] + pb[0, 0]
        xagg = _mlp3(aggr, w0[...], b0[...], w1[...], b1[...], w2[...], b2[...])
        nemb = xprep_r[...][0] + xagg
        nemb_o[...] = nemb.reshape(1, _PR, 128)
        s128 = (_hdot(sfull[...], nemb)
                + _hdot(slob[...], nemb * mlo128[...])
                + _hdot(shib[...], nemb * mhi128[...]))
        dsne_o[...] = _hdot(s128, p8[...]).reshape(1, _DBLK, D)
        nf = nf_r[...][0]
        s16 = (_hdot(sfull[...], nf)
               + _hdot(slob[...], nf * mlo16[...])
               + _hdot(shib[...], nf * mhi16[...]))
        dsnf_o[...] = _hdot(s16, p2[...]).reshape(1, _DBLK, 2)

    return pl.pallas_call(
        body,
        grid=(_NBLK,),
        in_specs=[
            pl.BlockSpec((1, 1, _PR, 128), lambda i: (0, i, 0, 0)),
            pl.BlockSpec((1, 1, _PR, 128), lambda i: (1, i, 0, 0)),
            pl.BlockSpec((1, _PR, 128), lambda i: (i, 0, 0)),
            pl.BlockSpec((1, _PR, 16), lambda i: (i, 0, 0)),
        ] + [_full_spec(a) for a in aa] + [_full_spec(a) for a in cs],
        out_specs=[
            pl.BlockSpec((1, _PR, 128), lambda i: (i, 0, 0)),
            pl.BlockSpec((1, _DBLK, D), lambda i: (i, 0, 0)),
            pl.BlockSpec((1, _DBLK, 2), lambda i: (i, 0, 0)),
        ],
        out_shape=[
            jax.ShapeDtypeStruct((_NBLK, _PR, 128), jnp.float32),
            jax.ShapeDtypeStruct((_NBLK, _DBLK, D), jnp.float32),
            jax.ShapeDtypeStruct((_NBLK, _DBLK, 2), jnp.float32),
        ],
    )(partials, partials, xprep, nf16, *aa, *cs)


def _dag_glob(dagfeat, dsne3, dsnf3, x00, dd_, gg):
    def body(df, dsne, dsnf, x0, dw0, db0, dw1, db1, dw2, db2,
             gw0, gb0, gw1, gb1, gw2, gb2, demb_o, glob_o):
        din = jnp.concatenate(
            [df[...], dsnf[...].reshape(G, 2), dsne[...].reshape(G, D)],
            axis=1)
        demb = _mlp3(din, dw0[...], db0[...], dw1[...], db1[...], dw2[...],
                     db2[...])
        demb_o[...] = demb.reshape(_NBLK, _DBLK, D)
        dagg = jnp.sum(demb, axis=0, keepdims=True)
        gin = jnp.concatenate([x0[...], dagg], axis=1)
        glob_o[...] = _mlp3(gin, gw0[...], gb0[...], gw1[...], gb1[...],
                            gw2[...], gb2[...])

    return pl.pallas_call(
        body,
        grid=(1,),
        in_specs=[_full_spec(a) for a in
                  (dagfeat, dsne3, dsnf3, x00, *dd_, *gg)],
        out_specs=[
            pl.BlockSpec((_NBLK, _DBLK, D), lambda i: (0, 0, 0)),
            pl.BlockSpec((1, D), lambda i: (0, 0)),
        ],
        out_shape=[
            jax.ShapeDtypeStruct((_NBLK, _DBLK, D), jnp.float32),
            jax.ShapeDtypeStruct((1, D), jnp.float32),
        ],
    )(dagfeat, dsne3, dsnf3, x00, *dd_, *gg)


_DGB = 200
_DGRID = G // _DGB
_DROWS = _DGB * W
_D3 = _DGB // _DBLK


def _dag_scores(demb3, glob, uu):
    def body(de, gl, w0, b0, w1, b1, w2, b2, out_o):
        drpt = jnp.broadcast_to(
            de[...].reshape(_DGB, D)[:, None, :],
            (_DGB, W, D)).reshape(_DROWS, D)
        gb = jnp.broadcast_to(gl[...], (_DROWS, D))
        wcol = (lax.broadcasted_iota(jnp.int32, (_DROWS, 1), 0) % W).astype(
            jnp.float32)
        din = jnp.concatenate([drpt, gb, wcol], axis=1)
        out_o[...] = _mlp3(din, w0[...], b0[...], w1[...], b1[...], w2[...],
                           b2[...])

    return pl.pallas_call(
        body,
        grid=(_DGRID,),
        in_specs=[pl.BlockSpec((_D3, _DBLK, D), lambda i: (i, 0, 0)),
                  _full_spec(glob)] + [_full_spec(a) for a in uu],
        out_specs=pl.BlockSpec((_DROWS, 1), lambda i: (i, 0)),
        out_shape=jax.ShapeDtypeStruct((G * W, 1), jnp.float32),
    )(demb3, glob, *uu)


def _node_scores(nemb, demb3, glob, vpk, cs):
    def body(ne, de, gl, w0a, w0b, w0g, b0, w1, b1, w2, b2,
             alls, his, m1, m2, out_o):
        de2 = de[...][0]
        q = jnp.concatenate([de2] * 8, axis=1)
        drpt = (_hdot(alls[...], q) * m1[...]
                + _hdot(his[...], q) * m2[...])
        glt = _bdot(gl[...], w0g[...]) + b0[...]
        glt = jnp.tile(glt, (1, 8))
        h = jnp.tanh(_bdot(ne[...][0], w0a[...]) + _bdot(drpt, w0b[...])
                     + glt)
        h = jnp.tanh(_bdot(h, w1[...]) + b1[...])
        out_o[...] = (_bdot(h, w2[...]) + b2[...]).reshape(1, _PR, 8)

    return pl.pallas_call(
        body,
        grid=(_NBLK,),
        in_specs=[
            pl.BlockSpec((1, _PR, 128), lambda i: (i, 0, 0)),
            pl.BlockSpec((1, _DBLK, D), lambda i: (i, 0, 0)),
            _full_spec(glob),
        ] + [_full_spec(a) for a in vpk] + [_full_spec(a) for a in cs],
        out_specs=pl.BlockSpec((1, _PR, 8), lambda i: (i, 0, 0)),
        out_shape=jax.ShapeDtypeStruct((_NBLK, _PR, 8), jnp.float32),
    )(nemb, demb3, glob, *vpk, *cs)


def _p(ps):
    return (ps[0], ps[1].reshape(1, -1), ps[2], ps[3].reshape(1, -1),
            ps[4], ps[5].reshape(1, -1))


def kernel(x, params, edge_index, ptr, batch):
    xp = x.reshape(_NBLK, _PR, 40)
    nf16 = x[:, 3:5].reshape(_NBLK, _PR, 16)
    x00 = x[0, 0].reshape(1, 1)
    dagfeat = x.reshape(G, NPD, 5)[:, 0, 1:3]

    src = edge_index[0]
    dst = edge_index[1]
    srcp = jnp.concatenate(
        [src, jnp.zeros((_EPAD - E,), jnp.int32)]).reshape(_EROWS, _LANES)
    dstp = jnp.concatenate(
        [dst, jnp.full((_EPAD - E,), N, jnp.int32)]).reshape(_EROWS, _LANES)
    zz = jnp.zeros((_ZROWS, D), jnp.float32)

    xprep, xproc = _prep_proc(xp, jnp.asarray(_SEL35),
                              _pk(params['prep']), _pk(params['proc']))
    partials = _edge_partials(xproc.reshape(N, D), srcp, dstp, zz)
    nemb, dsne3, dsnf3 = _agg_stage(
        partials[:, :N].reshape(2, _NBLK, _PR, 128), xprep, nf16,
        _pk(params['agg']),
        (jnp.asarray(_SFULL), jnp.asarray(_SLOB), jnp.asarray(_SHIB),
         jnp.asarray(_P8), jnp.asarray(_P2),
         jnp.asarray(_MLO128), jnp.asarray(_MHI128),
         jnp.asarray(_MLO16), jnp.asarray(_MHI16)))
    demb3, glob = _dag_glob(dagfeat, dsne3, dsnf3, x00, _p(params['dag']),
                            _p(params['glob']))
    dsc = _dag_scores(demb3, glob, _p(params['dag_score']))
    ns = params['node_score']
    vpk = (_bd8(ns[0][0:D]), _bd8(ns[0][D:2 * D]), ns[0][2 * D:3 * D],
           ns[1].reshape(1, -1), _bd8(ns[2]), _t8(ns[3]),
           _bd8(ns[4]), _t8(ns[5]))
    nsc = _node_scores(nemb, demb3, glob, vpk,
                       (jnp.asarray(_ALLS), jnp.asarray(_HIS),
                        jnp.asarray(_M1), jnp.asarray(_M2)))
    return nsc.reshape(N), dsc.reshape(G, W)


# speedup vs baseline: 24.4041x; 1.1069x over previous
import functools

import jax
import jax.numpy as jnp
from jax import lax
from jax.experimental import pallas as pl
from jax.experimental.pallas import tpu as pltpu
from jax.experimental.pallas import tpu_sc as plsc

N = 100000
E = 3200000
G = 1000
D = 16
W = 50
NPD = N // G

_LANES = 128
_CH = 6
_CE = _CH * _LANES
_TILES = 32
_ROWS_PER_TILE = 792
_STEPS = _ROWS_PER_TILE // _CH
_PAIRS = _STEPS // 2
_EROWS = _TILES * _ROWS_PER_TILE
_EPAD = _EROWS * _LANES
_ZROWS = 6256
_NACC = 16 * _ZROWS


def _sc_body(xproc, srcp, dstp, zz, out, acc, srcv, dstv, rows, gsem, ssem,
             isem):
    c = lax.axis_index("c")
    s = lax.axis_index("s")
    tid = c * 16 + s
    base = tid * _ROWS_PER_TILE
    pltpu.sync_copy(zz, acc.at[pl.ds(s * _ZROWS, _ZROWS)])
    plsc.subcore_barrier()

    def fire_gathers(b):
        for j in range(_CH):
            pltpu.async_copy(
                xproc.at[srcv.at[b, j]],
                rows.at[b, pl.ds(j * _LANES, _LANES)], gsem)

    def drain_gathers(b):
        for j in range(_CH):
            pltpu.make_async_copy(
                xproc.at[srcv.at[b, j]],
                rows.at[b, pl.ds(j * _LANES, _LANES)], gsem).wait()

    def fire_scatters(b):
        for j in range(_CH):
            pltpu.async_copy(
                rows.at[b, pl.ds(j * _LANES, _LANES)],
                acc.at[dstv.at[b, j]], ssem, add=True)

    def drain_scatters(b):
        for j in range(_CH):
            pltpu.make_async_copy(
                rows.at[b, pl.ds(j * _LANES, _LANES)],
                acc.at[dstv.at[b, j]], ssem).wait()

    def issue_idx(row, b):
        pltpu.async_copy(srcp.at[pl.ds(row, _CH)], srcv.at[b], isem)
        pltpu.async_copy(dstp.at[pl.ds(row, _CH)], dstv.at[b], isem)

    def wait_idx(b):
        pltpu.make_async_copy(srcp.at[pl.ds(0, _CH)], srcv.at[b], isem).wait()
        pltpu.make_async_copy(dstp.at[pl.ds(0, _CH)], dstv.at[b], isem).wait()

    pltpu.sync_copy(zz.at[pl.ds(0, _CE)], rows.at[1])
    pltpu.sync_copy(srcp.at[pl.ds(base + _CH, _CH)], srcv.at[1])
    pltpu.sync_copy(dstp.at[pl.ds(base + _CH, _CH)], dstv.at[1])
    fire_scatters(1)
    issue_idx(base, 0)

    def pair(i, carry):
        for b in (0, 1):
            g = 2 * i + b
            wait_idx(b)
            fire_gathers(b)
            drain_scatters(1 - b)
            nxt = jnp.minimum(g + 1, _STEPS - 1) * _CH
            issue_idx(base + nxt, 1 - b)
            drain_gathers(b)
            fire_scatters(b)
        return carry

    lax.fori_loop(0, _PAIRS, pair, 0)
    drain_scatters(1)
    wait_idx(0)
    plsc.subcore_barrier()
    pltpu.sync_copy(
        acc.at[pl.ds(s * _ZROWS, _ZROWS)], out.at[c, pl.ds(s * _ZROWS, _ZROWS)]
    )


def _edge_partials(xproc, srcp, dstp, zz):
    f = pl.kernel(
        _sc_body,
        out_type=jax.ShapeDtypeStruct((2, _NACC, D), jnp.float32),
        mesh=plsc.VectorSubcoreMesh(core_axis_name="c", subcore_axis_name="s"),
        scratch_types=[
            pltpu.VMEM_SHARED((_NACC, D), jnp.float32),
            pltpu.VMEM((2, _CH, _LANES), jnp.int32),
            pltpu.VMEM((2, _CH, _LANES), jnp.int32),
            pltpu.VMEM((2, _CE, D), jnp.float32),
            pltpu.SemaphoreType.DMA,
            pltpu.SemaphoreType.DMA,
            pltpu.SemaphoreType.DMA,
        ],
        compiler_params=pltpu.CompilerParams(use_tc_tiling_on_sc=False),
    )
    return f(xproc, srcp, dstp, zz)


_R = 4000
_NBLK = N // _R
_DBLK = _R // NPD


def _bdot(a, b):
    return jnp.dot(a.astype(jnp.bfloat16), b.astype(jnp.bfloat16),
                   preferred_element_type=jnp.float32)


def _mlp3(h, w0, b0, w1, b1, w2, b2):
    h = jnp.tanh(_bdot(h, w0) + b0)
    h = jnp.tanh(_bdot(h, w1) + b1)
    return _bdot(h, w2) + b2


def _full_spec(a):
    r = a.ndim
    return pl.BlockSpec(a.shape, lambda i, _r=r: (0,) * _r)


def _prep_proc(nf, pp, qq):
    def body(nf_r, pw0, pb0, pw1, pb1, pw2, pb2, qw0, qb0, qw1, qb1, qw2, qb2,
             prep_o, proc_o):
        xprep = _mlp3(nf_r[...], pw0[...], pb0[...], pw1[...], pb1[...],
                      pw2[...], pb2[...])
        prep_o[...] = xprep
        proc_o[...] = _mlp3(xprep, qw0[...], qb0[...], qw1[...], qb1[...],
                            qw2[...], qb2[...])

    return pl.pallas_call(
        body,
        grid=(_NBLK,),
        in_specs=[pl.BlockSpec((_R, 2), lambda i: (i, 0))]
        + [_full_spec(a) for a in pp] + [_full_spec(a) for a in qq],
        out_specs=[pl.BlockSpec((_R, D), lambda i: (i, 0))] * 2,
        out_shape=[jax.ShapeDtypeStruct((N, D), jnp.float32)] * 2,
    )(nf, *pp, *qq)


def _agg_stage(partials, xprep, nf, aa):
    def body(pa, pb, xprep_r, nf_r, w0, b0, w1, b1, w2, b2, nemb_o, dsum_o):
        aggr = pa[0] + pb[0]
        xagg = _mlp3(aggr, w0[...], b0[...], w1[...], b1[...], w2[...], b2[...])
        nemb = xprep_r[...] + xagg
        nemb_o[...] = nemb
        ncomb = jnp.concatenate([nf_r[...], nemb], axis=1)
        dsum = jnp.sum(ncomb.reshape(_DBLK, NPD, D + 2), axis=1)
        dsum_o[...] = dsum.reshape(1, _DBLK, D + 2)

    return pl.pallas_call(
        body,
        grid=(_NBLK,),
        in_specs=[
            pl.BlockSpec((1, _R, D), lambda i: (0, i, 0)),
            pl.BlockSpec((1, _R, D), lambda i: (1, i, 0)),
            pl.BlockSpec((_R, D), lambda i: (i, 0)),
            pl.BlockSpec((_R, 2), lambda i: (i, 0)),
        ] + [_full_spec(a) for a in aa],
        out_specs=[
            pl.BlockSpec((_R, D), lambda i: (i, 0)),
            pl.BlockSpec((1, _DBLK, D + 2), lambda i: (i, 0, 0)),
        ],
        out_shape=[
            jax.ShapeDtypeStruct((N, D), jnp.float32),
            jax.ShapeDtypeStruct((_NBLK, _DBLK, D + 2), jnp.float32),
        ],
    )(partials, partials, xprep, nf, *aa)


def _dag_glob(dagfeat, dagsum, x00, dd, gg):
    def body(df, ds_, x0, dw0, db0, dw1, db1, dw2, db2,
             gw0, gb0, gw1, gb1, gw2, gb2, demb_o, glob_o):
        din = jnp.concatenate([df[...], ds_[...]], axis=1)
        demb = _mlp3(din, dw0[...], db0[...], dw1[...], db1[...], dw2[...],
                     db2[...])
        demb_o[...] = demb
        dagg = jnp.sum(demb, axis=0, keepdims=True)
        gin = jnp.concatenate([x0[...], dagg], axis=1)
        glob_o[...] = _mlp3(gin, gw0[...], gb0[...], gw1[...], gb1[...],
                            gw2[...], gb2[...])

    return pl.pallas_call(
        body,
        grid=(1,),
        in_specs=[_full_spec(a) for a in
                  (dagfeat, dagsum, x00, *dd, *gg)],
        out_specs=[
            pl.BlockSpec((G, D), lambda i: (0, 0)),
            pl.BlockSpec((1, D), lambda i: (0, 0)),
        ],
        out_shape=[
            jax.ShapeDtypeStruct((G, D), jnp.float32),
            jax.ShapeDtypeStruct((1, D), jnp.float32),
        ],
    )(dagfeat, dagsum, x00, *dd, *gg)


_DGB = 200
_DGRID = G // _DGB
_DROWS = _DGB * W


def _dag_scores(demb, glob, uu):
    def body(de, gl, w0, b0, w1, b1, w2, b2, out_o):
        drpt = jnp.broadcast_to(
            de[...][:, None, :], (_DGB, W, D)).reshape(_DROWS, D)
        gb = jnp.broadcast_to(gl[...], (_DROWS, D))
        wcol = (lax.broadcasted_iota(jnp.int32, (_DROWS, 1), 0) % W).astype(
            jnp.float32)
        din = jnp.concatenate([drpt, gb, wcol], axis=1)
        out_o[...] = _mlp3(din, w0[...], b0[...], w1[...], b1[...], w2[...],
                           b2[...])

    return pl.pallas_call(
        body,
        grid=(_DGRID,),
        in_specs=[pl.BlockSpec((_DGB, D), lambda i: (i, 0)),
                  _full_spec(glob)] + [_full_spec(a) for a in uu],
        out_specs=pl.BlockSpec((_DROWS, 1), lambda i: (i, 0)),
        out_shape=jax.ShapeDtypeStruct((G * W, 1), jnp.float32),
    )(demb, glob, *uu)


def _node_scores(nemb, demb3, glob, vv):
    def body(ne, de, gl, w0, b0, w1, b1, w2, b2, out_o):
        drpt = jnp.broadcast_to(
            de[...][0][:, None, :], (_DBLK, NPD, D)).reshape(_R, D)
        gb = jnp.broadcast_to(gl[...], (_R, D))
        nin = jnp.concatenate([ne[...], drpt, gb], axis=1)
        out_o[...] = _mlp3(nin, w0[...], b0[...], w1[...], b1[...], w2[...],
                           b2[...])

    return pl.pallas_call(
        body,
        grid=(_NBLK,),
        in_specs=[
            pl.BlockSpec((_R, D), lambda i: (i, 0)),
            pl.BlockSpec((1, _DBLK, D), lambda i: (i, 0, 0)),
            _full_spec(glob),
        ] + [_full_spec(a) for a in vv],
        out_specs=pl.BlockSpec((_R, 1), lambda i: (i, 0)),
        out_shape=jax.ShapeDtypeStruct((N, 1), jnp.float32),
    )(nemb, demb3, glob, *vv)


def _p(ps):
    return (ps[0], ps[1].reshape(1, -1), ps[2], ps[3].reshape(1, -1),
            ps[4], ps[5].reshape(1, -1))


def kernel(x, params, edge_index, ptr, batch):
    nf = x[:, 3:5]
    x00 = x[0, 0].reshape(1, 1)
    dagfeat = x.reshape(G, NPD, 5)[:, 0, 1:3]

    src = edge_index[0]
    dst = edge_index[1]
    srcp = jnp.concatenate(
        [src, jnp.zeros((_EPAD - E,), jnp.int32)]).reshape(_EROWS, _LANES)
    dstp = jnp.concatenate(
        [dst, jnp.full((_EPAD - E,), N, jnp.int32)]).reshape(_EROWS, _LANES)
    zz = jnp.zeros((_ZROWS, D), jnp.float32)

    xprep, xproc = _prep_proc(nf, _p(params['prep']), _p(params['proc']))
    partials = _edge_partials(xproc, srcp, dstp, zz)
    nemb, dagsum3 = _agg_stage(partials, xprep, nf, _p(params['agg']))
    demb, glob = _dag_glob(dagfeat, dagsum3.reshape(G, D + 2), x00,
                           _p(params['dag']), _p(params['glob']))
    dsc = _dag_scores(demb, glob, _p(params['dag_score']))
    nsc = _node_scores(nemb, demb.reshape(_NBLK, _DBLK, D), glob,
                       _p(params['node_score']))
    return nsc[:, 0], dsc.reshape(G, W)
